# baseline (speedup 1.0000x reference)
import functools

import jax
import jax.numpy as jnp
from jax import lax
from jax.experimental import pallas as pl
from jax.experimental.pallas import tpu as pltpu
from jax.experimental.pallas import tpu_sc as plsc

VOCAB = 100000
EMB = 32
HIDDEN = 512
CTX = 20
BATCH = 1024

BN = 2048


def _sc_gather(table, idx_flat, n_rows):
    info = plsc.get_sparse_core_info()
    nw = info.num_cores * info.num_subcores
    b_per_w = n_rows // nw
    mesh = plsc.VectorSubcoreMesh(core_axis_name="c", subcore_axis_name="s")

    @functools.partial(
        pl.kernel,
        mesh=mesh,
        compiler_params=pltpu.CompilerParams(use_tc_tiling_on_sc=False),
        out_type=jax.ShapeDtypeStruct((n_rows, EMB), jnp.float32),
        scratch_types=[
            pltpu.VMEM((b_per_w,), jnp.int32),
            pltpu.VMEM((b_per_w, EMB), jnp.float32),
            pltpu.SemaphoreType.DMA,
        ],
    )
    def gather_k(idx_hbm, table_hbm, out_hbm, idx_v, rows_v, sem):
        wid = lax.axis_index("s") * info.num_cores + lax.axis_index("c")
        base = wid * b_per_w
        pltpu.sync_copy(idx_hbm.at[pl.ds(base, b_per_w)], idx_v)
        pltpu.async_copy(table_hbm.at[idx_v], rows_v, sem).wait()
        pltpu.sync_copy(rows_v, out_hbm.at[pl.ds(base, b_per_w)])

    return gather_k(idx_flat, table)


def _mlp_body(flat_ref, w1_ref, b1_ref, w2_ref, b2_ref, out_ref, h_ref):
    @pl.when(pl.program_id(0) == 0)
    def _():
        a = flat_ref[...].astype(jnp.bfloat16)
        w1 = w1_ref[...].astype(jnp.bfloat16)
        hh = jnp.dot(a, w1, preferred_element_type=jnp.float32)
        h_ref[...] = jnp.maximum(hh + b1_ref[...], 0.0).astype(jnp.bfloat16)

    w2 = w2_ref[...].astype(jnp.bfloat16)
    acc = jnp.dot(h_ref[...], w2, preferred_element_type=jnp.float32)
    out_ref[...] = acc + b2_ref[...]


def kernel(x, emb_table, W1, b1, W2, b2):
    idx_flat = x.reshape(-1).astype(jnp.int32)
    flat = _sc_gather(emb_table, idx_flat, BATCH * CTX)
    flat = flat.reshape(BATCH, CTX * EMB)

    nblocks = pl.cdiv(VOCAB, BN)
    logits = pl.pallas_call(
        _mlp_body,
        grid=(nblocks,),
        in_specs=[
            pl.BlockSpec((BATCH, CTX * EMB), lambda j: (0, 0)),
            pl.BlockSpec((CTX * EMB, HIDDEN), lambda j: (0, 0)),
            pl.BlockSpec((1, HIDDEN), lambda j: (0, 0)),
            pl.BlockSpec((HIDDEN, BN), lambda j: (0, j)),
            pl.BlockSpec((1, BN), lambda j: (0, j)),
        ],
        out_specs=pl.BlockSpec((BATCH, BN), lambda j: (0, j)),
        out_shape=jax.ShapeDtypeStruct((BATCH, VOCAB), jnp.float32),
        scratch_shapes=[pltpu.VMEM((BATCH, HIDDEN), jnp.bfloat16)],
        compiler_params=pltpu.CompilerParams(
            dimension_semantics=("arbitrary",),
        ),
    )(flat, W1, b1.reshape(1, HIDDEN), W2, b2.reshape(1, VOCAB))

    return logits
